# baseline (speedup 1.0000x reference)
import jax
import jax.numpy as jnp
from jax.experimental import pallas as pl


def kernel(x, symbol_library):
    batch, seq_len, dim = x.shape
    S = 2048

    def body(tab_ref, out_ref):
        out_ref[...] = tab_ref[...][None]

    return pl.pallas_call(
        body,
        grid=(seq_len // S, batch),
        in_specs=[pl.BlockSpec((S, dim), lambda i, b: (i, 0))],
        out_specs=pl.BlockSpec((1, S, dim), lambda i, b: (b, i, 0)),
        out_shape=jax.ShapeDtypeStruct((batch, seq_len, dim), jnp.float32),
    )(symbol_library)
